# baseline (speedup 1.0000x reference)
import functools

import jax
import jax.numpy as jnp
from jax import lax
from jax.experimental import pallas as pl
from jax.experimental.pallas import tpu as pltpu
from jax.experimental.pallas import tpu_sc as plsc

_ROWS = 100000
_DIM = 64
_NWORKERS = 32
_CHUNK = 400
_NCHUNKS = _ROWS // _CHUNK
_ROUNDS = -(-_NCHUNKS // _NWORKERS)


def _sc_copy(w_hbm, out_hbm, bufs, in_sems, out_sems):
    wid = lax.axis_index("s") * 2 + lax.axis_index("c")

    def has(i):
        return wid + i * _NWORKERS < _NCHUNKS

    def in_cp(i, b):
        sl = pl.ds((wid + i * _NWORKERS) * _CHUNK, _CHUNK)
        return pltpu.make_async_copy(w_hbm.at[sl, :], bufs.at[b], in_sems.at[b])

    def out_cp(i, b):
        sl = pl.ds((wid + i * _NWORKERS) * _CHUNK, _CHUNK)
        return pltpu.make_async_copy(bufs.at[b], out_hbm.at[sl, :], out_sems.at[b])

    for i in range(2):
        @pl.when(has(i))
        def _(i=i):
            in_cp(i, i % 2).start()

    for i in range(_ROUNDS):
        b = i % 2

        @pl.when(has(i))
        def _(i=i, b=b):
            in_cp(i, b).wait()
            out_cp(i, b).start()

        j = i + 2
        if j < _ROUNDS:
            @pl.when(has(j))
            def _(i=i, j=j, b=b):
                out_cp(i, b).wait()
                in_cp(j, b).start()

    for i in range(_ROUNDS):
        @pl.when(jnp.logical_and(has(i), jnp.logical_not(has(i + 2))))
        def _(i=i):
            out_cp(i, i % 2).wait()


def kernel(graph, feat, edge_feat, emb_weight):
    n, d = emb_weight.shape
    run = functools.partial(
        pl.kernel,
        mesh=plsc.VectorSubcoreMesh(core_axis_name="c", subcore_axis_name="s"),
        out_type=jax.ShapeDtypeStruct((n, d), emb_weight.dtype),
        scratch_types=[
            pltpu.VMEM((2, _CHUNK, _DIM), jnp.float32),
            pltpu.SemaphoreType.DMA((2,)),
            pltpu.SemaphoreType.DMA((2,)),
        ],
    )(_sc_copy)
    return run(emb_weight)


# speedup vs baseline: 1.1691x; 1.1691x over previous
import jax
import jax.numpy as jnp
from jax.experimental import pallas as pl
from jax.experimental.pallas import tpu as pltpu

_ROWS = 100000
_DIM = 64


def _copy(w_ref, o_ref, buf, in_sem, out_sem):
    pltpu.make_async_copy(w_ref, buf, in_sem).start()
    pltpu.make_async_copy(w_ref, buf, in_sem).wait()
    pltpu.make_async_copy(buf, o_ref, out_sem).start()
    pltpu.make_async_copy(buf, o_ref, out_sem).wait()


def kernel(graph, feat, edge_feat, emb_weight):
    n, d = emb_weight.shape
    return pl.pallas_call(
        _copy,
        in_specs=[pl.BlockSpec(memory_space=pl.ANY)],
        out_specs=pl.BlockSpec(memory_space=pl.ANY),
        out_shape=jax.ShapeDtypeStruct((n, d), emb_weight.dtype),
        scratch_shapes=[
            pltpu.VMEM((_ROWS, _DIM), jnp.float32),
            pltpu.SemaphoreType.DMA,
            pltpu.SemaphoreType.DMA,
        ],
        compiler_params=pltpu.CompilerParams(
            skip_device_barrier=True,
            disable_bounds_checks=True,
            disable_semaphore_checks=True,
        ),
    )(emb_weight)
